# Initial kernel scaffold; baseline (speedup 1.0000x reference)
import functools

import jax
import jax.numpy as jnp
from jax import lax
from jax.experimental import pallas as pl
from jax.experimental.pallas import tpu as pltpu
from jax.experimental.pallas import tpu_sc as plsc

N = 10000
D = 128
E = 320000
NC = 2
NS = 16
NW = NC * NS
E_PER_W = E // NW
CHUNK = 80
N_CHUNKS = E_PER_W // CHUNK
ROWS_PER_TILE = N // NS


def _sc_segment_sum(h, src, dst, zeros):
    @functools.partial(
        pl.kernel,
        out_type=jax.ShapeDtypeStruct((NC, N, D), jnp.float32),
        mesh=plsc.VectorSubcoreMesh(core_axis_name="c", subcore_axis_name="s"),
        scratch_types=[
            pltpu.VMEM((CHUNK,), jnp.int32),
            pltpu.VMEM((CHUNK,), jnp.int32),
            pltpu.VMEM((CHUNK, D), jnp.float32),
            pltpu.VMEM_SHARED((N, D), jnp.float32),
            pltpu.SemaphoreType.DMA,
        ],
    )
    def k(h_hbm, src_hbm, dst_hbm, z_hbm, out_hbm, sidx, didx, rows, agg, sem):
        c = lax.axis_index("c")
        s = lax.axis_index("s")
        wid = s * NC + c
        pltpu.sync_copy(z_hbm.at[pl.ds(s * ROWS_PER_TILE, ROWS_PER_TILE)],
                        agg.at[pl.ds(s * ROWS_PER_TILE, ROWS_PER_TILE)])
        plsc.subcore_barrier()

        base = wid * E_PER_W

        @pl.loop(0, N_CHUNKS)
        def _(j):
            off = base + j * CHUNK
            pltpu.sync_copy(src_hbm.at[pl.ds(off, CHUNK)], sidx)
            pltpu.sync_copy(dst_hbm.at[pl.ds(off, CHUNK)], didx)
            pltpu.async_copy(h_hbm.at[sidx], rows, sem).wait()
            pltpu.sync_copy(rows, agg.at[didx], add=True)

        plsc.subcore_barrier()
        pltpu.sync_copy(agg.at[pl.ds(s * ROWS_PER_TILE, ROWS_PER_TILE)],
                        out_hbm.at[c, pl.ds(s * ROWS_PER_TILE, ROWS_PER_TILE)])

    return k(h, src, dst, zeros)


def _tc_linear_relu(p0, p1, h, W, b2d):
    def body(p0_ref, p1_ref, h_ref, w_ref, b_ref, o_ref):
        acc = p0_ref[...] + p1_ref[...] + h_ref[...]
        y = lax.dot_general(acc, w_ref[...], (((1,), (1,)), ((), ())),
                            preferred_element_type=jnp.float32)
        o_ref[...] = jnp.maximum(y + b_ref[...], 0.0)

    return pl.pallas_call(
        body,
        out_shape=jax.ShapeDtypeStruct((N, D), jnp.float32),
    )(p0, p1, h, W, b2d)


def kernel(x, g, W0, b0, W1, b1, W2, b2):
    src = g[0]
    dst = g[1]
    zeros = jnp.zeros_like(x)
    h = x
    for W, b in ((W0, b0), (W1, b1), (W2, b2)):
        parts = _sc_segment_sum(h, src, dst, zeros)
        h = _tc_linear_relu(parts[0], parts[1], h, W, b.reshape(1, D))
    return h


# speedup vs baseline: 4.9631x; 4.9631x over previous
import functools

import jax
import jax.numpy as jnp
from jax import lax
from jax.experimental import pallas as pl
from jax.experimental.pallas import tpu as pltpu
from jax.experimental.pallas import tpu_sc as plsc

N = 10000
D = 128
E = 320000
NC = 2
NS = 16
NW = NC * NS
E_PER_W = E // NW
CHUNK = 80
N_CHUNKS = E_PER_W // CHUNK
STRIPE = 624
LAST_STRIPE = N - 15 * STRIPE


def _sc_segment_sum(h, src, dst, zeros):
    @functools.partial(
        pl.kernel,
        out_type=jax.ShapeDtypeStruct((NC, N, D), jnp.float32),
        mesh=plsc.VectorSubcoreMesh(core_axis_name="c", subcore_axis_name="s"),
        scratch_types=[
            pltpu.VMEM((CHUNK,), jnp.int32),
            pltpu.VMEM((CHUNK,), jnp.int32),
            pltpu.VMEM((CHUNK, D), jnp.float32),
            pltpu.VMEM_SHARED((N, D), jnp.float32),
            pltpu.SemaphoreType.DMA,
        ],
    )
    def k(h_hbm, src_hbm, dst_hbm, z_hbm, out_hbm, sidx, didx, rows, agg, sem):
        c = lax.axis_index("c")
        s = lax.axis_index("s")
        wid = s * NC + c

        def stripe_copy(mk_src, mk_dst):
            @pl.when(s < NS - 1)
            def _():
                sl = pl.ds(s * STRIPE, STRIPE)
                pltpu.sync_copy(mk_src(sl), mk_dst(sl))

            @pl.when(s == NS - 1)
            def _():
                sl = pl.ds((NS - 1) * STRIPE, LAST_STRIPE)
                pltpu.sync_copy(mk_src(sl), mk_dst(sl))

        stripe_copy(lambda sl: z_hbm.at[sl], lambda sl: agg.at[sl])
        plsc.subcore_barrier()

        base = wid * E_PER_W

        @pl.loop(0, N_CHUNKS)
        def _(j):
            off = base + j * CHUNK
            pltpu.sync_copy(src_hbm.at[pl.ds(off, CHUNK)], sidx)
            pltpu.sync_copy(dst_hbm.at[pl.ds(off, CHUNK)], didx)
            pltpu.async_copy(h_hbm.at[sidx], rows, sem).wait()
            pltpu.sync_copy(rows, agg.at[didx], add=True)

        plsc.subcore_barrier()
        stripe_copy(lambda sl: agg.at[sl], lambda sl: out_hbm.at[c, sl])

    return k(h, src, dst, zeros)


def _tc_linear_relu(p0, p1, h, W, b2d):
    def body(p0_ref, p1_ref, h_ref, w_ref, b_ref, o_ref):
        acc = p0_ref[...] + p1_ref[...] + h_ref[...]
        y = lax.dot_general(acc, w_ref[...], (((1,), (1,)), ((), ())),
                            preferred_element_type=jnp.float32)
        o_ref[...] = jnp.maximum(y + b_ref[...], 0.0)

    return pl.pallas_call(
        body,
        out_shape=jax.ShapeDtypeStruct((N, D), jnp.float32),
    )(p0, p1, h, W, b2d)


def kernel(x, g, W0, b0, W1, b1, W2, b2):
    src = g[0]
    dst = g[1]
    zeros = jnp.zeros_like(x)
    h = x
    for W, b in ((W0, b0), (W1, b1), (W2, b2)):
        parts = _sc_segment_sum(h, src, dst, zeros)
        h = _tc_linear_relu(parts[0], parts[1], h, W, b.reshape(1, D))
    return h


# speedup vs baseline: 12.8956x; 2.5983x over previous
import functools

import jax
import jax.numpy as jnp
from jax import lax
from jax.experimental import pallas as pl
from jax.experimental.pallas import tpu as pltpu
from jax.experimental.pallas import tpu_sc as plsc

N = 10000
D = 128
HD = D // 2
E = 320000
NC = 2
NS = 16
E_PER_SUB = E // NS
CHUNK = 80
N_CHUNKS = E_PER_SUB // CHUNK
NBUF = 5
STRIPE = 624
LAST_STRIPE = N - (NS - 1) * STRIPE


def _sc_segment_sum(h0, h1, src, dst, zeros):
    @functools.partial(
        pl.kernel,
        out_type=jax.ShapeDtypeStruct((NC, N, HD), jnp.float32),
        mesh=plsc.VectorSubcoreMesh(core_axis_name="c", subcore_axis_name="s"),
        scratch_types=[
            pltpu.VMEM((N_CHUNKS, CHUNK), jnp.int32),
            pltpu.VMEM((N_CHUNKS, CHUNK), jnp.int32),
            pltpu.VMEM((NBUF, CHUNK, HD), jnp.float32),
            pltpu.VMEM_SHARED((N, HD), jnp.float32),
            [pltpu.SemaphoreType.DMA] * NBUF,
        ],
        compiler_params=pltpu.CompilerParams(use_tc_tiling_on_sc=False),
    )
    def k(h0_hbm, h1_hbm, src_hbm, dst_hbm, z_hbm, out_hbm,
          sidx, didx, rows, agg, sems):
        c = lax.axis_index("c")
        s = lax.axis_index("s")

        def stripe_copy(mk_src, mk_dst):
            @pl.when(s < NS - 1)
            def _():
                sl = pl.ds(s * STRIPE, STRIPE)
                pltpu.sync_copy(mk_src(sl), mk_dst(sl))

            @pl.when(s == NS - 1)
            def _():
                sl = pl.ds((NS - 1) * STRIPE, LAST_STRIPE)
                pltpu.sync_copy(mk_src(sl), mk_dst(sl))

        pltpu.sync_copy(src_hbm.at[s], sidx)
        pltpu.sync_copy(dst_hbm.at[s], didx)
        stripe_copy(lambda sl: z_hbm.at[sl], lambda sl: agg.at[sl])
        plsc.subcore_barrier()

        def accumulate(h_hbm):
            def gather(chunk, b):
                pltpu.async_copy(h_hbm.at[sidx.at[chunk]], rows.at[b], sems[b])

            def gather_wait(chunk, b):
                pltpu.make_async_copy(h_hbm.at[sidx.at[chunk]], rows.at[b],
                                      sems[b]).wait()

            for b in range(NBUF):
                gather(b, b)

            @pl.loop(0, N_CHUNKS, step=NBUF)
            def _(j):
                for b in range(NBUF):
                    cur = j + b
                    gather_wait(cur, b)
                    pltpu.sync_copy(rows.at[b], agg.at[didx.at[cur]], add=True)
                    nxt = cur + NBUF

                    @pl.when(nxt < N_CHUNKS)
                    def _():
                        gather(nxt, b)

        @pl.when(c == 0)
        def _():
            accumulate(h0_hbm)

        @pl.when(c == 1)
        def _():
            accumulate(h1_hbm)

        plsc.subcore_barrier()
        stripe_copy(lambda sl: agg.at[sl], lambda sl: out_hbm.at[c, sl])

    return k(h0, h1, src, dst, zeros)


def _tc_linear_relu(parts, h0, h1, W, b2d, last):
    def body(p_ref, h0_ref, h1_ref, w_ref, b_ref, *o_refs):
        acc0 = p_ref[0] + h0_ref[...]
        acc1 = p_ref[1] + h1_ref[...]
        dn = (((1,), (1,)), ((), ()))
        y = lax.dot_general(acc0, w_ref[:, :HD], dn,
                            preferred_element_type=jnp.float32)
        y += lax.dot_general(acc1, w_ref[:, HD:], dn,
                             preferred_element_type=jnp.float32)
        y = jnp.maximum(y + b_ref[...], 0.0)
        if last:
            o_refs[0][...] = y
        else:
            o_refs[0][...] = y[:, :HD]
            o_refs[1][...] = y[:, HD:]

    out_shape = (jax.ShapeDtypeStruct((N, D), jnp.float32) if last else
                 [jax.ShapeDtypeStruct((N, HD), jnp.float32)] * 2)
    return pl.pallas_call(
        body,
        out_shape=out_shape,
    )(parts, h0, h1, W, b2d)


def kernel(x, g, W0, b0, W1, b1, W2, b2):
    src = g[0].reshape(NS, N_CHUNKS, CHUNK)
    dst = g[1].reshape(NS, N_CHUNKS, CHUNK)
    zeros = jnp.zeros((N, HD), dtype=x.dtype)
    h0, h1 = x[:, :HD], x[:, HD:]
    layers = ((W0, b0), (W1, b1), (W2, b2))
    for i, (W, b) in enumerate(layers):
        last = i == len(layers) - 1
        parts = _sc_segment_sum(h0, h1, src, dst, zeros)
        out = _tc_linear_relu(parts, h0, h1, W, b.reshape(1, D), last)
        if last:
            return out
        h0, h1 = out
